# baseline (speedup 1.0000x reference)
import functools

import jax
import jax.numpy as jnp
from jax import lax
from jax.experimental import pallas as pl
from jax.experimental.pallas import tpu as pltpu
from jax.experimental.pallas import tpu_sc as plsc

N = 10000
N_PAD = 10240
CHK = N_PAD // 16
DUMMY = N
NC, NS = 2, 16
NW = NC * NS
EPR = 128
RPW = 76
NBUF = 4
R_TOTAL = NW * RPW
E_PAD = R_TOTAL * EPR
HID = 128
D2 = 64
DA = 16


def _make_edge_agg(feat_dim):
    mesh = plsc.VectorSubcoreMesh(
        core_axis_name="c", subcore_axis_name="s", num_cores=NC, num_subcores=NS
    )

    @functools.partial(
        pl.kernel,
        out_type=jax.ShapeDtypeStruct((NC, N_PAD, feat_dim), jnp.float32),
        mesh=mesh,
        compiler_params=pltpu.CompilerParams(use_tc_tiling_on_sc=False),
        scratch_types=[
            pltpu.VMEM((RPW, EPR), jnp.int32),
            pltpu.VMEM((RPW, EPR), jnp.int32),
            [pltpu.VMEM((EPR, feat_dim), jnp.float32)] * NBUF,
            pltpu.VMEM_SHARED((N_PAD, feat_dim), jnp.float32),
            [pltpu.SemaphoreType.DMA] * NBUF,
        ],
    )
    def agg(src_hbm, dst_hbm, tab_hbm, zero_hbm, out_hbm,
            src_v, dst_v, rows_v, acc_sh, gsem):
        c = lax.axis_index("c")
        s = lax.axis_index("s")
        w = c * NS + s
        pltpu.sync_copy(src_hbm.at[w], src_v)
        pltpu.sync_copy(dst_hbm.at[w], dst_v)
        pltpu.sync_copy(zero_hbm.at[pl.ds(s * CHK, CHK)],
                        acc_sh.at[pl.ds(s * CHK, CHK)])

        def _route(i, carry):
            for j in range(EPR // 16):
                sl = pl.ds(j * 16, 16)
                sv = src_v[i, sl]
                dv = dst_v[i, sl]
                dst_v[i, sl] = jnp.where(
                    sv == dv, jnp.full((16,), DUMMY, jnp.int32), dv)
            return carry

        lax.fori_loop(0, RPW, _route, 0)
        plsc.subcore_barrier()

        def _edge_block(o, carry):
            i0 = o * NBUF
            descs = [
                pltpu.async_copy(tab_hbm.at[src_v.at[i0 + b]],
                                 rows_v[b], gsem[b])
                for b in range(NBUF)
            ]
            for b in range(NBUF):
                descs[b].wait()
                pltpu.sync_copy(rows_v[b], acc_sh.at[dst_v.at[i0 + b]],
                                add=True)
            return carry

        lax.fori_loop(0, RPW // NBUF, _edge_block, 0)
        plsc.subcore_barrier()
        pltpu.sync_copy(acc_sh.at[pl.ds(s * CHK, CHK)],
                        out_hbm.at[c].at[pl.ds(s * CHK, CHK)])

    return agg


_agg_a = _make_edge_agg(DA)
_agg_b = _make_edge_agg(D2)


def _dense1(accA, pos8, W1l, b1, W1r, W2l, W2r):
    def body(acc_ref, pos_ref, w1l_ref, b1_ref, w1r_ref, w2l_ref, w2r_ref,
             z_ref, hr_ref):
        acc = acc_ref[0] + acc_ref[1]
        cnt = jnp.maximum(acc[:, 3:4], 1.0)
        mean1 = acc[:, 0:3] / cnt
        pos = pos_ref[:, 0:3]
        h1 = jnp.maximum(
            jnp.dot(mean1, w1l_ref[...], preferred_element_type=jnp.float32)
            + b1_ref[...]
            + jnp.dot(pos, w1r_ref[...], preferred_element_type=jnp.float32),
            0.0)
        z_ref[...] = jnp.dot(h1, w2l_ref[...],
                             preferred_element_type=jnp.float32)
        hr_ref[...] = jnp.dot(h1, w2r_ref[...],
                              preferred_element_type=jnp.float32)

    return pl.pallas_call(
        body,
        out_shape=(jax.ShapeDtypeStruct((N_PAD, D2), jnp.float32),
                   jax.ShapeDtypeStruct((N_PAD, D2), jnp.float32)),
    )(accA, pos8, W1l, b1.reshape(1, HID), W1r, W2l, W2r)


def _dense2(accB, accA, hr, b2, Wdp, bdp):
    def body(accb_ref, acca_ref, hr_ref, b2_ref, wd_ref, bd_ref,
             p_ref, am_ref):
        acc2 = accb_ref[0] + accb_ref[1]
        cnt = jnp.maximum(acca_ref[0][:, 3:4] + acca_ref[1][:, 3:4], 1.0)
        h2 = jnp.maximum(acc2 / cnt + b2_ref[...] + hr_ref[...], 0.0)
        rows = lax.broadcasted_iota(jnp.int32, (N_PAD, 1), 0)
        h2 = jnp.where(rows < N, h2, 0.0)
        g = jnp.sum(h2, axis=0, keepdims=True) * (1.0 / N)
        logits = (jnp.dot(g, wd_ref[...], preferred_element_type=jnp.float32)
                  + bd_ref[...])
        m = jnp.max(logits, axis=-1, keepdims=True)
        e = jnp.exp(logits - m)
        p = e / jnp.sum(e, axis=-1, keepdims=True)
        p_ref[...] = p
        lane = lax.broadcasted_iota(jnp.int32, (1, 16), 1)
        first_max = jnp.min(jnp.where(logits >= m, lane, 16), axis=-1,
                            keepdims=True)
        am_ref[...] = first_max.astype(jnp.int32)

    return pl.pallas_call(
        body,
        out_shape=(jax.ShapeDtypeStruct((1, 16), jnp.float32),
                   jax.ShapeDtypeStruct((1, 1), jnp.int32)),
    )(accB, accA, hr, b2.reshape(1, D2), Wdp, bdp)


def kernel(pos, face, W1l, b1, W1r, W2l, b2, W2r, Wd, bd):
    f = face.astype(jnp.int32)
    src = jnp.concatenate([f[0], f[1], f[0]])
    dst = jnp.concatenate([f[1], f[2], f[2]])
    pad = E_PAD - src.shape[0]
    src3d = jnp.pad(src, (0, pad)).reshape(NW, RPW, EPR)
    dst3d = jnp.pad(dst, (0, pad)).reshape(NW, RPW, EPR)

    pos16 = (jnp.zeros((N_PAD, DA), jnp.float32)
             .at[:N, 0:3].set(pos).at[:N, 3].set(1.0))
    zeroA = jnp.zeros((N_PAD, DA), jnp.float32)
    zeroB = jnp.zeros((N_PAD, D2), jnp.float32)

    accA = _agg_a(src3d, dst3d, pos16, zeroA)

    pos8 = jnp.zeros((N_PAD, 8), jnp.float32).at[:N, 0:3].set(pos)
    z, hr = _dense1(accA, pos8, W1l, b1, W1r, W2l, W2r)

    accB = _agg_b(src3d, dst3d, z, zeroB)

    Wdp = jnp.zeros((D2, 16), jnp.float32).at[:, :10].set(Wd)
    bdp = jnp.full((1, 16), -1e30, jnp.float32).at[0, :10].set(bd)
    p_pad, am = _dense2(accB, accA, hr, b2, Wdp, bdp)
    return (p_pad[0, :10], am[0, 0])


# speedup vs baseline: 2.2491x; 2.2491x over previous
import functools

import jax
import jax.numpy as jnp
from jax import lax
from jax.experimental import pallas as pl
from jax.experimental.pallas import tpu as pltpu
from jax.experimental.pallas import tpu_sc as plsc

N = 10000
N_PAD = 10240
CHK = N_PAD // 16
DUMMY = N
NC, NS = 2, 16
NW = NC * NS
EPR = 128
RPW = 76
NBUF = 4
R_TOTAL = NW * RPW
E_PAD = R_TOTAL * EPR
HID = 128
D2 = 64
DA = 16


def _make_edge_agg(feat_dim):
    mesh = plsc.VectorSubcoreMesh(
        core_axis_name="c", subcore_axis_name="s", num_cores=NC, num_subcores=NS
    )

    @functools.partial(
        pl.kernel,
        out_type=jax.ShapeDtypeStruct((NC, N_PAD, feat_dim), jnp.float32),
        mesh=mesh,
        compiler_params=pltpu.CompilerParams(use_tc_tiling_on_sc=False),
        scratch_types=[
            pltpu.VMEM((RPW, EPR), jnp.int32),
            pltpu.VMEM((RPW, EPR), jnp.int32),
            [pltpu.VMEM((EPR, feat_dim), jnp.float32)] * NBUF,
            pltpu.VMEM_SHARED((N_PAD, feat_dim), jnp.float32),
            [pltpu.SemaphoreType.DMA] * NBUF,
        ],
    )
    def agg(src_hbm, dst_hbm, tab_hbm, zero_hbm, out_hbm,
            src_v, dst_v, rows_v, acc_sh, gsem):
        c = lax.axis_index("c")
        s = lax.axis_index("s")
        w = c * NS + s
        pltpu.sync_copy(src_hbm.at[w], src_v)
        pltpu.sync_copy(dst_hbm.at[w], dst_v)
        pltpu.sync_copy(zero_hbm.at[pl.ds(s * CHK, CHK)],
                        acc_sh.at[pl.ds(s * CHK, CHK)])

        def _route(i, carry):
            for j in range(EPR // 16):
                sl = pl.ds(j * 16, 16)
                sv = src_v[i, sl]
                dv = dst_v[i, sl]
                dst_v[i, sl] = jnp.where(
                    sv == dv, jnp.full((16,), DUMMY, jnp.int32), dv)
            return carry

        lax.fori_loop(0, RPW, _route, 0)
        plsc.subcore_barrier()

        def _edge_block(o, carry):
            i0 = o * NBUF
            descs = [
                pltpu.async_copy(tab_hbm.at[src_v.at[i0 + b]],
                                 rows_v[b], gsem[b])
                for b in range(NBUF)
            ]
            for b in range(NBUF):
                descs[b].wait()
                pltpu.sync_copy(rows_v[b], acc_sh.at[dst_v.at[i0 + b]],
                                add=True)
            return carry

        lax.fori_loop(0, RPW // NBUF, _edge_block, 0)
        plsc.subcore_barrier()
        pltpu.sync_copy(acc_sh.at[pl.ds(s * CHK, CHK)],
                        out_hbm.at[c].at[pl.ds(s * CHK, CHK)])

    return agg


_agg_a = _make_edge_agg(DA)
_agg_b = _make_edge_agg(D2)


def _dense1(accA, pos8, W1l, b1, W1r, W2l, W2r):
    def body(acc_ref, pos_ref, w1l_ref, b1_ref, w1r_ref, w2l_ref, w2r_ref,
             z_ref, hr_ref):
        acc = acc_ref[0] + acc_ref[1]
        cnt = jnp.maximum(acc[:, 3:4], 1.0)
        mean1 = acc[:, 0:3] / cnt
        pos = pos_ref[:, 0:3]
        h1 = jnp.maximum(
            jnp.dot(mean1, w1l_ref[...], preferred_element_type=jnp.float32)
            + b1_ref[...]
            + jnp.dot(pos, w1r_ref[...], preferred_element_type=jnp.float32),
            0.0)
        z_ref[...] = jnp.dot(h1, w2l_ref[...],
                             preferred_element_type=jnp.float32)
        hr_ref[...] = jnp.dot(h1, w2r_ref[...],
                              preferred_element_type=jnp.float32)

    return pl.pallas_call(
        body,
        out_shape=(jax.ShapeDtypeStruct((N_PAD, D2), jnp.float32),
                   jax.ShapeDtypeStruct((N_PAD, D2), jnp.float32)),
    )(accA, pos8, W1l, b1.reshape(1, HID), W1r, W2l, W2r)


def _dense2(accB, accA, hr, b2, Wdp, bdp):
    def body(accb_ref, acca_ref, hr_ref, b2_ref, wd_ref, bd_ref,
             p_ref, am_ref):
        acc2 = accb_ref[0] + accb_ref[1]
        cnt = jnp.maximum(acca_ref[0][:, 3:4] + acca_ref[1][:, 3:4], 1.0)
        h2 = jnp.maximum(acc2 / cnt + b2_ref[...] + hr_ref[...], 0.0)
        rows = lax.broadcasted_iota(jnp.int32, (N_PAD, 1), 0)
        h2 = jnp.where(rows < N, h2, 0.0)
        g = jnp.sum(h2, axis=0, keepdims=True) * (1.0 / N)
        logits = (jnp.dot(g, wd_ref[...], preferred_element_type=jnp.float32)
                  + bd_ref[...])
        m = jnp.max(logits, axis=-1, keepdims=True)
        e = jnp.exp(logits - m)
        p = e / jnp.sum(e, axis=-1, keepdims=True)
        p_ref[...] = p
        lane = lax.broadcasted_iota(jnp.int32, (1, 16), 1)
        first_max = jnp.min(jnp.where(logits >= m, lane, 16), axis=-1,
                            keepdims=True)
        am_ref[...] = first_max.astype(jnp.int32)

    return pl.pallas_call(
        body,
        out_shape=(jax.ShapeDtypeStruct((1, 16), jnp.float32),
                   jax.ShapeDtypeStruct((1, 1), jnp.int32)),
    )(accB, accA, hr, b2.reshape(1, D2), Wdp, bdp)


def kernel(pos, face, W1l, b1, W1r, W2l, b2, W2r, Wd, bd):
    f = face.astype(jnp.int32)
    src = jnp.concatenate([f[0], f[1], f[0]])
    dst = jnp.concatenate([f[1], f[2], f[2]])
    pad = E_PAD - src.shape[0]
    pad_idx = jnp.arange(pad, dtype=jnp.int32)
    src3d = jnp.concatenate([src, pad_idx % N]).reshape(NW, RPW, EPR)
    dst3d = jnp.concatenate(
        [dst, N + 1 + pad_idx % (N_PAD - N - 1)]).reshape(NW, RPW, EPR)

    pos16 = (jnp.zeros((N_PAD, DA), jnp.float32)
             .at[:N, 0:3].set(pos).at[:N, 3].set(1.0))
    zeroA = jnp.zeros((N_PAD, DA), jnp.float32)
    zeroB = jnp.zeros((N_PAD, D2), jnp.float32)

    accA = _agg_a(src3d, dst3d, pos16, zeroA)

    pos8 = jnp.zeros((N_PAD, 8), jnp.float32).at[:N, 0:3].set(pos)
    z, hr = _dense1(accA, pos8, W1l, b1, W1r, W2l, W2r)

    accB = _agg_b(src3d, dst3d, z, zeroB)

    Wdp = jnp.zeros((D2, 16), jnp.float32).at[:, :10].set(Wd)
    bdp = jnp.full((1, 16), -1e30, jnp.float32).at[0, :10].set(bd)
    p_pad, am = _dense2(accB, accA, hr, b2, Wdp, bdp)
    return (p_pad[0, :10], am[0, 0])


# speedup vs baseline: 2.3269x; 1.0346x over previous
import functools

import jax
import jax.numpy as jnp
from jax import lax
from jax.experimental import pallas as pl
from jax.experimental.pallas import tpu as pltpu
from jax.experimental.pallas import tpu_sc as plsc

N = 10000
N_PAD = 10240
CHK = N_PAD // 16
DUMMY = N
NC, NS = 2, 16
NW = NC * NS
EPR = 128
RPW = 76
NBUF = 4
R_TOTAL = NW * RPW
E_PAD = R_TOTAL * EPR
HID = 128
D2 = 64
DA = 16


def _make_edge_agg(feat_dim):
    mesh = plsc.VectorSubcoreMesh(
        core_axis_name="c", subcore_axis_name="s", num_cores=NC, num_subcores=NS
    )

    @functools.partial(
        pl.kernel,
        out_type=jax.ShapeDtypeStruct((NC, N_PAD, feat_dim), jnp.float32),
        mesh=mesh,
        compiler_params=pltpu.CompilerParams(use_tc_tiling_on_sc=False),
        scratch_types=[
            pltpu.VMEM((RPW, EPR), jnp.int32),
            pltpu.VMEM((RPW, EPR), jnp.int32),
            [pltpu.VMEM((EPR, feat_dim), jnp.float32)] * NBUF,
            pltpu.VMEM_SHARED((N_PAD, feat_dim), jnp.float32),
            [pltpu.SemaphoreType.DMA] * NBUF,
            [pltpu.SemaphoreType.DMA] * NBUF,
        ],
    )
    def agg(src_hbm, dst_hbm, tab_hbm, zero_hbm, out_hbm,
            src_v, dst_v, rows_v, acc_sh, gsem, ssem):
        c = lax.axis_index("c")
        s = lax.axis_index("s")
        w = c * NS + s
        pltpu.sync_copy(src_hbm.at[w], src_v)
        pltpu.sync_copy(dst_hbm.at[w], dst_v)
        pltpu.sync_copy(zero_hbm.at[pl.ds(s * CHK, CHK)],
                        acc_sh.at[pl.ds(s * CHK, CHK)])

        def _route(i, carry):
            for j in range(EPR // 16):
                sl = pl.ds(j * 16, 16)
                sv = src_v[i, sl]
                dv = dst_v[i, sl]
                dst_v[i, sl] = jnp.where(
                    sv == dv, jnp.full((16,), DUMMY, jnp.int32), dv)
            return carry

        lax.fori_loop(0, RPW, _route, 0)
        plsc.subcore_barrier()

        def _edge_block(o, carry):
            i0 = o * NBUF
            gds = [
                pltpu.async_copy(tab_hbm.at[src_v.at[i0 + b]],
                                 rows_v[b], gsem[b])
                for b in range(NBUF)
            ]
            sds = []
            for b in range(NBUF):
                gds[b].wait()
                sds.append(
                    pltpu.async_copy(rows_v[b], acc_sh.at[dst_v.at[i0 + b]],
                                     ssem[b], add=True))
            for d in sds:
                d.wait()
            return carry

        lax.fori_loop(0, RPW // NBUF, _edge_block, 0)
        plsc.subcore_barrier()
        pltpu.sync_copy(acc_sh.at[pl.ds(s * CHK, CHK)],
                        out_hbm.at[c].at[pl.ds(s * CHK, CHK)])

    return agg


_agg_a = _make_edge_agg(DA)
_agg_b = _make_edge_agg(D2)


def _dense1(accA, pos8, W1l, b1, W1r, W2l, W2r):
    def body(acc_ref, pos_ref, w1l_ref, b1_ref, w1r_ref, w2l_ref, w2r_ref,
             z_ref, hr_ref):
        acc = acc_ref[0] + acc_ref[1]
        cnt = jnp.maximum(acc[:, 3:4], 1.0)
        mean1 = acc[:, 0:3] / cnt
        pos = pos_ref[:, 0:3]
        h1 = jnp.maximum(
            jnp.dot(mean1, w1l_ref[...], preferred_element_type=jnp.float32)
            + b1_ref[...]
            + jnp.dot(pos, w1r_ref[...], preferred_element_type=jnp.float32),
            0.0)
        z_ref[...] = jnp.dot(h1, w2l_ref[...],
                             preferred_element_type=jnp.float32)
        hr_ref[...] = jnp.dot(h1, w2r_ref[...],
                              preferred_element_type=jnp.float32)

    return pl.pallas_call(
        body,
        out_shape=(jax.ShapeDtypeStruct((N_PAD, D2), jnp.float32),
                   jax.ShapeDtypeStruct((N_PAD, D2), jnp.float32)),
    )(accA, pos8, W1l, b1.reshape(1, HID), W1r, W2l, W2r)


def _dense2(accB, accA, hr, b2, Wdp, bdp):
    def body(accb_ref, acca_ref, hr_ref, b2_ref, wd_ref, bd_ref,
             p_ref, am_ref):
        acc2 = accb_ref[0] + accb_ref[1]
        cnt = jnp.maximum(acca_ref[0][:, 3:4] + acca_ref[1][:, 3:4], 1.0)
        h2 = jnp.maximum(acc2 / cnt + b2_ref[...] + hr_ref[...], 0.0)
        rows = lax.broadcasted_iota(jnp.int32, (N_PAD, 1), 0)
        h2 = jnp.where(rows < N, h2, 0.0)
        g = jnp.sum(h2, axis=0, keepdims=True) * (1.0 / N)
        logits = (jnp.dot(g, wd_ref[...], preferred_element_type=jnp.float32)
                  + bd_ref[...])
        m = jnp.max(logits, axis=-1, keepdims=True)
        e = jnp.exp(logits - m)
        p = e / jnp.sum(e, axis=-1, keepdims=True)
        p_ref[...] = p
        lane = lax.broadcasted_iota(jnp.int32, (1, 16), 1)
        first_max = jnp.min(jnp.where(logits >= m, lane, 16), axis=-1,
                            keepdims=True)
        am_ref[...] = first_max.astype(jnp.int32)

    return pl.pallas_call(
        body,
        out_shape=(jax.ShapeDtypeStruct((1, 16), jnp.float32),
                   jax.ShapeDtypeStruct((1, 1), jnp.int32)),
    )(accB, accA, hr, b2.reshape(1, D2), Wdp, bdp)


def kernel(pos, face, W1l, b1, W1r, W2l, b2, W2r, Wd, bd):
    f = face.astype(jnp.int32)
    src = jnp.concatenate([f[0], f[1], f[0]])
    dst = jnp.concatenate([f[1], f[2], f[2]])
    pad = E_PAD - src.shape[0]
    pad_idx = jnp.arange(pad, dtype=jnp.int32)
    src3d = jnp.concatenate([src, pad_idx % N]).reshape(NW, RPW, EPR)
    dst3d = jnp.concatenate(
        [dst, N + 1 + pad_idx % (N_PAD - N - 1)]).reshape(NW, RPW, EPR)

    pos16 = (jnp.zeros((N_PAD, DA), jnp.float32)
             .at[:N, 0:3].set(pos).at[:N, 3].set(1.0))
    zeroA = jnp.zeros((N_PAD, DA), jnp.float32)
    zeroB = jnp.zeros((N_PAD, D2), jnp.float32)

    accA = _agg_a(src3d, dst3d, pos16, zeroA)

    pos8 = jnp.zeros((N_PAD, 8), jnp.float32).at[:N, 0:3].set(pos)
    z, hr = _dense1(accA, pos8, W1l, b1, W1r, W2l, W2r)

    accB = _agg_b(src3d, dst3d, z, zeroB)

    Wdp = jnp.zeros((D2, 16), jnp.float32).at[:, :10].set(Wd)
    bdp = jnp.full((1, 16), -1e30, jnp.float32).at[0, :10].set(bd)
    p_pad, am = _dense2(accB, accA, hr, b2, Wdp, bdp)
    return (p_pad[0, :10], am[0, 0])


# speedup vs baseline: 2.4188x; 1.0395x over previous
import functools

import jax
import jax.numpy as jnp
from jax import lax
from jax.experimental import pallas as pl
from jax.experimental.pallas import tpu as pltpu
from jax.experimental.pallas import tpu_sc as plsc

N = 10000
N_PAD = 10240
CHK = N_PAD // 16
DUMMY = N
NC, NS = 2, 16
NW = NC * NS
EPR = 128
RPW = 80
NBUF = 8
R_TOTAL = NW * RPW
E_PAD = R_TOTAL * EPR
HID = 128
D2 = 64
DA = 16


def _make_edge_agg(feat_dim):
    mesh = plsc.VectorSubcoreMesh(
        core_axis_name="c", subcore_axis_name="s", num_cores=NC, num_subcores=NS
    )

    @functools.partial(
        pl.kernel,
        out_type=jax.ShapeDtypeStruct((NC, N_PAD, feat_dim), jnp.float32),
        mesh=mesh,
        compiler_params=pltpu.CompilerParams(use_tc_tiling_on_sc=False),
        scratch_types=[
            pltpu.VMEM((RPW, EPR), jnp.int32),
            pltpu.VMEM((RPW, EPR), jnp.int32),
            [pltpu.VMEM((EPR, feat_dim), jnp.float32)] * NBUF,
            pltpu.VMEM_SHARED((N_PAD, feat_dim), jnp.float32),
            [pltpu.SemaphoreType.DMA] * NBUF,
            [pltpu.SemaphoreType.DMA] * NBUF,
        ],
    )
    def agg(src_hbm, dst_hbm, tab_hbm, zero_hbm, out_hbm,
            src_v, dst_v, rows_v, acc_sh, gsem, ssem):
        c = lax.axis_index("c")
        s = lax.axis_index("s")
        w = c * NS + s
        pltpu.sync_copy(src_hbm.at[w], src_v)
        pltpu.sync_copy(dst_hbm.at[w], dst_v)
        pltpu.sync_copy(zero_hbm.at[pl.ds(s * CHK, CHK)],
                        acc_sh.at[pl.ds(s * CHK, CHK)])

        def _route(i, carry):
            for j in range(EPR // 16):
                sl = pl.ds(j * 16, 16)
                sv = src_v[i, sl]
                dv = dst_v[i, sl]
                dst_v[i, sl] = jnp.where(
                    sv == dv, jnp.full((16,), DUMMY, jnp.int32), dv)
            return carry

        lax.fori_loop(0, RPW, _route, 0)
        plsc.subcore_barrier()

        def _edge_block(o, carry):
            i0 = o * NBUF
            gds = [
                pltpu.async_copy(tab_hbm.at[src_v.at[i0 + b]],
                                 rows_v[b], gsem[b])
                for b in range(NBUF)
            ]
            sds = []
            for b in range(NBUF):
                gds[b].wait()
                sds.append(
                    pltpu.async_copy(rows_v[b], acc_sh.at[dst_v.at[i0 + b]],
                                     ssem[b], add=True))
            for d in sds:
                d.wait()
            return carry

        lax.fori_loop(0, RPW // NBUF, _edge_block, 0)
        plsc.subcore_barrier()
        pltpu.sync_copy(acc_sh.at[pl.ds(s * CHK, CHK)],
                        out_hbm.at[c].at[pl.ds(s * CHK, CHK)])

    return agg


_agg_a = _make_edge_agg(DA)
_agg_b = _make_edge_agg(D2)


def _dense1(accA, pos8, W1l, b1, W1r, W2l, W2r):
    def body(acc_ref, pos_ref, w1l_ref, b1_ref, w1r_ref, w2l_ref, w2r_ref,
             z_ref, hr_ref):
        acc = acc_ref[0] + acc_ref[1]
        cnt = jnp.maximum(acc[:, 3:4], 1.0)
        mean1 = acc[:, 0:3] / cnt
        pos = pos_ref[:, 0:3]
        h1 = jnp.maximum(
            jnp.dot(mean1, w1l_ref[...], preferred_element_type=jnp.float32)
            + b1_ref[...]
            + jnp.dot(pos, w1r_ref[...], preferred_element_type=jnp.float32),
            0.0)
        z_ref[...] = jnp.dot(h1, w2l_ref[...],
                             preferred_element_type=jnp.float32)
        hr_ref[...] = jnp.dot(h1, w2r_ref[...],
                              preferred_element_type=jnp.float32)

    return pl.pallas_call(
        body,
        out_shape=(jax.ShapeDtypeStruct((N_PAD, D2), jnp.float32),
                   jax.ShapeDtypeStruct((N_PAD, D2), jnp.float32)),
    )(accA, pos8, W1l, b1.reshape(1, HID), W1r, W2l, W2r)


def _dense2(accB, accA, hr, b2, Wdp, bdp):
    def body(accb_ref, acca_ref, hr_ref, b2_ref, wd_ref, bd_ref,
             p_ref, am_ref):
        acc2 = accb_ref[0] + accb_ref[1]
        cnt = jnp.maximum(acca_ref[0][:, 3:4] + acca_ref[1][:, 3:4], 1.0)
        h2 = jnp.maximum(acc2 / cnt + b2_ref[...] + hr_ref[...], 0.0)
        rows = lax.broadcasted_iota(jnp.int32, (N_PAD, 1), 0)
        h2 = jnp.where(rows < N, h2, 0.0)
        g = jnp.sum(h2, axis=0, keepdims=True) * (1.0 / N)
        logits = (jnp.dot(g, wd_ref[...], preferred_element_type=jnp.float32)
                  + bd_ref[...])
        m = jnp.max(logits, axis=-1, keepdims=True)
        e = jnp.exp(logits - m)
        p = e / jnp.sum(e, axis=-1, keepdims=True)
        p_ref[...] = p
        lane = lax.broadcasted_iota(jnp.int32, (1, 16), 1)
        first_max = jnp.min(jnp.where(logits >= m, lane, 16), axis=-1,
                            keepdims=True)
        am_ref[...] = first_max.astype(jnp.int32)

    return pl.pallas_call(
        body,
        out_shape=(jax.ShapeDtypeStruct((1, 16), jnp.float32),
                   jax.ShapeDtypeStruct((1, 1), jnp.int32)),
    )(accB, accA, hr, b2.reshape(1, D2), Wdp, bdp)


def kernel(pos, face, W1l, b1, W1r, W2l, b2, W2r, Wd, bd):
    f = face.astype(jnp.int32)
    src = jnp.concatenate([f[0], f[1], f[0]])
    dst = jnp.concatenate([f[1], f[2], f[2]])
    pad = E_PAD - src.shape[0]
    pad_idx = jnp.arange(pad, dtype=jnp.int32)
    src3d = jnp.concatenate([src, pad_idx % N]).reshape(NW, RPW, EPR)
    dst3d = jnp.concatenate(
        [dst, N + 1 + pad_idx % (N_PAD - N - 1)]).reshape(NW, RPW, EPR)

    pos16 = (jnp.zeros((N_PAD, DA), jnp.float32)
             .at[:N, 0:3].set(pos).at[:N, 3].set(1.0))
    zeroA = jnp.zeros((N_PAD, DA), jnp.float32)
    zeroB = jnp.zeros((N_PAD, D2), jnp.float32)

    accA = _agg_a(src3d, dst3d, pos16, zeroA)

    pos8 = jnp.zeros((N_PAD, 8), jnp.float32).at[:N, 0:3].set(pos)
    z, hr = _dense1(accA, pos8, W1l, b1, W1r, W2l, W2r)

    accB = _agg_b(src3d, dst3d, z, zeroB)

    Wdp = jnp.zeros((D2, 16), jnp.float32).at[:, :10].set(Wd)
    bdp = jnp.full((1, 16), -1e30, jnp.float32).at[0, :10].set(bd)
    p_pad, am = _dense2(accB, accA, hr, b2, Wdp, bdp)
    return (p_pad[0, :10], am[0, 0])
